# baseline (device time: 31892 ns/iter reference)
import jax
import jax.numpy as jnp
from jax import lax
from jax.experimental import pallas as pl
from jax.experimental.pallas import tpu as pltpu

N_DEV = 4
N_LAYERS = 3
N_EXCH = 4 * N_LAYERS
HB = 512
N_CHUNK = 8


def kernel(x, Win0, Wout0, Win1, Wout1, Win2, Wout2):
    b, d = x.shape
    h_per = Win0.shape[1]

    def body(x_ref, win0_hbm, wout0_hbm, win1_hbm, wout1_hbm, win2_hbm,
             wout2_hbm, out_ref, win_buf, wout_buf, send_buf, recv_buf,
             win_sems, wout_sems, send_sems, recv_sems):
        my = lax.axis_index("i")
        p1 = my ^ 1
        p2 = 3 - my

        wins_hbm = [win0_hbm, win1_hbm, win2_hbm]
        wouts_hbm = [wout0_hbm, wout1_hbm, wout2_hbm]

        def win_chunks(layer):
            rows = d // N_CHUNK
            for c in range(N_CHUNK):
                yield pltpu.make_async_copy(
                    wins_hbm[layer].at[pl.ds(c * rows, rows)],
                    win_buf.at[layer, pl.ds(c * rows, rows)],
                    win_sems.at[layer])

        def wout_chunks(layer):
            rows = h_per // N_CHUNK
            for c in range(N_CHUNK):
                yield pltpu.make_async_copy(
                    wouts_hbm[layer].at[pl.ds(c * rows, rows)],
                    wout_buf.at[layer, pl.ds(c * rows, rows)],
                    wout_sems.at[layer])

        def wait_win(layer):
            for cp in win_chunks(layer):
                cp.wait()

        def wait_wout(layer):
            for cp in wout_chunks(layer):
                cp.wait()

        def exch(e, partner):
            return pltpu.make_async_remote_copy(
                src_ref=send_buf.at[e],
                dst_ref=recv_buf.at[e],
                send_sem=send_sems.at[e],
                recv_sem=recv_sems.at[e],
                device_id=(partner,),
                device_id_type=pl.DeviceIdType.MESH,
            )

        def partner_of(stage, half, a, bdev):
            return (a if half == 0 else bdev) if stage == 0 else \
                   (bdev if half == 0 else a)

        for layer in range(N_LAYERS):
            for cp in win_chunks(layer):
                cp.start()
            for cp in wout_chunks(layer):
                cp.start()

        barrier_sem = pltpu.get_barrier_semaphore()
        for nbr in (p1, p2):
            pl.semaphore_signal(
                barrier_sem, inc=1,
                device_id=(nbr,), device_id_type=pl.DeviceIdType.MESH,
            )
        pl.semaphore_wait(barrier_sem, 2)

        x_b = x_ref[:, :].astype(jnp.bfloat16)
        rows0 = d // N_CHUNK
        hn = None
        for c, cp in enumerate(win_chunks(0)):
            cp.wait()
            wk = win_buf[0, c * rows0:(c + 1) * rows0, :].astype(jnp.bfloat16)
            pk = jnp.dot(x_b[:, c * rows0:(c + 1) * rows0], wk,
                         preferred_element_type=jnp.float32)
            hn = pk if hn is None else hn + pk
        h = jnp.maximum(hn, 0.0).astype(jnp.bfloat16)

        for layer in range(N_LAYERS):
            base = 4 * layer
            wait_wout(layer)

            for half in range(2):
                w_out_h = wout_buf[layer, :, half * HB:(half + 1) * HB
                                   ].astype(jnp.bfloat16)
                acc_h = jnp.dot(h, w_out_h, preferred_element_type=jnp.float32)
                e = base + half
                send_buf[e, :, :] = acc_h.astype(jnp.bfloat16)
                exch(e, partner_of(0, half, p1, p2)).start()

            for half in range(2):
                e0 = base + half
                exch(e0, partner_of(0, half, p1, p2)).wait_recv()
                e1 = base + 2 + half
                send_buf[e1, :, :] = send_buf[e0, :, :] + recv_buf[e0, :, :]
                exch(e1, partner_of(1, half, p1, p2)).start()

            if layer < N_LAYERS - 1:
                wait_win(layer + 1)
                w_next = win_buf[layer + 1].astype(jnp.bfloat16)
                e10 = base + 2
                exch(e10, partner_of(1, 0, p1, p2)).wait_recv()
                x0 = send_buf[e10, :, :] + recv_buf[e10, :, :]
                hn = jnp.dot(x0, w_next[0:HB, :],
                             preferred_element_type=jnp.float32)
                e11 = base + 3
                exch(e11, partner_of(1, 1, p1, p2)).wait_recv()
                x1 = send_buf[e11, :, :] + recv_buf[e11, :, :]
                hn = hn + jnp.dot(x1, w_next[HB:2 * HB, :],
                                  preferred_element_type=jnp.float32)
                h = jnp.maximum(hn, 0.0).astype(jnp.bfloat16)
            else:
                for half in range(2):
                    e1 = base + 2 + half
                    exch(e1, partner_of(1, half, p1, p2)).wait_recv()
                    out_ref[:, half * HB:(half + 1) * HB] = (
                        send_buf[e1, :, :] + recv_buf[e1, :, :]
                    ).astype(jnp.float32)

        for layer in range(N_LAYERS):
            for stage in range(2):
                for half in range(2):
                    e = 4 * layer + 2 * stage + half
                    exch(e, partner_of(stage, half, p1, p2)).wait_send()

    return pl.pallas_call(
        body,
        out_shape=jax.ShapeDtypeStruct((b, d), jnp.float32),
        in_specs=[pl.BlockSpec(memory_space=pltpu.VMEM)]
        + [pl.BlockSpec(memory_space=pl.ANY)] * 6,
        out_specs=pl.BlockSpec(memory_space=pltpu.VMEM),
        scratch_shapes=[
            pltpu.VMEM((N_LAYERS, d, h_per), jnp.float32),
            pltpu.VMEM((N_LAYERS, h_per, d), jnp.float32),
            pltpu.VMEM((N_EXCH, b, HB), jnp.bfloat16),
            pltpu.VMEM((N_EXCH, b, HB), jnp.bfloat16),
            pltpu.SemaphoreType.DMA((N_LAYERS,)),
            pltpu.SemaphoreType.DMA((N_LAYERS,)),
            pltpu.SemaphoreType.DMA((N_EXCH,)),
            pltpu.SemaphoreType.DMA((N_EXCH,)),
        ],
        compiler_params=pltpu.CompilerParams(
            collective_id=0, vmem_limit_bytes=110 * 1024 * 1024
        ),
    )(x, Win0, Wout0, Win1, Wout1, Win2, Wout2)


# device time: 31462 ns/iter; 1.0137x vs baseline; 1.0137x over previous
import jax
import jax.numpy as jnp
from jax import lax
from jax.experimental import pallas as pl
from jax.experimental.pallas import tpu as pltpu

N_DEV = 4
N_LAYERS = 3
N_EXCH = 4 * N_LAYERS
HB = 512
N_CHUNK = 8


def kernel(x, Win0, Wout0, Win1, Wout1, Win2, Wout2):
    b, d = x.shape
    h_per = Win0.shape[1]

    def body(x_ref, win0_hbm, wout0_hbm, win1_hbm, wout1_hbm, win2_hbm,
             wout2_hbm, out_ref, win_buf, wout_buf, send_buf, recv_buf,
             win_sems, wout_sems, send_sems, recv_sems):
        my = lax.axis_index("i")
        p1 = my ^ 1
        p2 = 3 - my

        wins_hbm = [win0_hbm, win1_hbm, win2_hbm]
        wouts_hbm = [wout0_hbm, wout1_hbm, wout2_hbm]

        def win_chunks(layer):
            rows = d // N_CHUNK
            for c in range(N_CHUNK):
                yield pltpu.make_async_copy(
                    wins_hbm[layer].at[pl.ds(c * rows, rows)],
                    win_buf.at[layer, pl.ds(c * rows, rows)],
                    win_sems.at[layer])

        def wout_chunks(layer):
            rows = h_per // N_CHUNK
            for c in range(N_CHUNK):
                yield pltpu.make_async_copy(
                    wouts_hbm[layer].at[pl.ds(c * rows, rows)],
                    wout_buf.at[layer, pl.ds(c * rows, rows)],
                    wout_sems.at[layer])

        def wait_win(layer):
            for cp in win_chunks(layer):
                cp.wait()

        def wait_wout(layer):
            for cp in wout_chunks(layer):
                cp.wait()

        def exch(e, partner):
            return pltpu.make_async_remote_copy(
                src_ref=send_buf.at[e],
                dst_ref=recv_buf.at[e],
                send_sem=send_sems.at[e],
                recv_sem=recv_sems.at[e],
                device_id=(partner,),
                device_id_type=pl.DeviceIdType.MESH,
            )

        def partner_of(stage, half, a, bdev):
            return (a if half == 0 else bdev) if stage == 0 else \
                   (bdev if half == 0 else a)

        for layer in range(N_LAYERS):
            for cp in win_chunks(layer):
                cp.start()
            for cp in wout_chunks(layer):
                cp.start()

        barrier_sem = pltpu.get_barrier_semaphore()
        for nbr in (p1, p2):
            pl.semaphore_signal(
                barrier_sem, inc=1,
                device_id=(nbr,), device_id_type=pl.DeviceIdType.MESH,
            )
        pl.semaphore_wait(barrier_sem, 2)

        x_b = x_ref[:, :].astype(jnp.bfloat16)
        rows0 = d // N_CHUNK
        hn = None
        for c, cp in enumerate(win_chunks(0)):
            cp.wait()
            wk = win_buf[0, c * rows0:(c + 1) * rows0, :].astype(jnp.bfloat16)
            pk = jnp.dot(x_b[:, c * rows0:(c + 1) * rows0], wk,
                         preferred_element_type=jnp.float32)
            hn = pk if hn is None else hn + pk
        h = jnp.maximum(hn, 0.0).astype(jnp.bfloat16)

        for layer in range(N_LAYERS):
            base = 4 * layer
            wait_wout(layer)

            w_out = wout_buf[layer].astype(jnp.bfloat16)

            acc = [None, None]
            for half in range(2):
                acc[half] = jnp.dot(h, w_out[:, half * HB:(half + 1) * HB],
                                    preferred_element_type=jnp.float32)
                e = base + half
                send_buf[e, :, :] = acc[half].astype(jnp.bfloat16)
                exch(e, partner_of(0, half, p1, p2)).start()

            for half in range(2):
                e0 = base + half
                exch(e0, partner_of(0, half, p1, p2)).wait_recv()
                acc[half] = acc[half] + recv_buf[e0, :, :].astype(jnp.float32)
                e1 = base + 2 + half
                send_buf[e1, :, :] = acc[half].astype(jnp.bfloat16)
                exch(e1, partner_of(1, half, p1, p2)).start()

            if layer < N_LAYERS - 1:
                wait_win(layer + 1)
                w_next = win_buf[layer + 1].astype(jnp.bfloat16)
                e10 = base + 2
                exch(e10, partner_of(1, 0, p1, p2)).wait_recv()
                x0 = (acc[0] + recv_buf[e10, :, :].astype(jnp.float32)
                      ).astype(jnp.bfloat16)
                hn = jnp.dot(x0, w_next[0:HB, :],
                             preferred_element_type=jnp.float32)
                e11 = base + 3
                exch(e11, partner_of(1, 1, p1, p2)).wait_recv()
                x1 = (acc[1] + recv_buf[e11, :, :].astype(jnp.float32)
                      ).astype(jnp.bfloat16)
                hn = hn + jnp.dot(x1, w_next[HB:2 * HB, :],
                                  preferred_element_type=jnp.float32)
                h = jnp.maximum(hn, 0.0).astype(jnp.bfloat16)
            else:
                for half in range(2):
                    e1 = base + 2 + half
                    exch(e1, partner_of(1, half, p1, p2)).wait_recv()
                    out_ref[:, half * HB:(half + 1) * HB] = (
                        acc[half] + recv_buf[e1, :, :].astype(jnp.float32))

        for layer in range(N_LAYERS):
            for stage in range(2):
                for half in range(2):
                    e = 4 * layer + 2 * stage + half
                    exch(e, partner_of(stage, half, p1, p2)).wait_send()

    return pl.pallas_call(
        body,
        out_shape=jax.ShapeDtypeStruct((b, d), jnp.float32),
        in_specs=[pl.BlockSpec(memory_space=pltpu.VMEM)]
        + [pl.BlockSpec(memory_space=pl.ANY)] * 6,
        out_specs=pl.BlockSpec(memory_space=pltpu.VMEM),
        scratch_shapes=[
            pltpu.VMEM((N_LAYERS, d, h_per), jnp.float32),
            pltpu.VMEM((N_LAYERS, h_per, d), jnp.float32),
            pltpu.VMEM((N_EXCH, b, HB), jnp.bfloat16),
            pltpu.VMEM((N_EXCH, b, HB), jnp.bfloat16),
            pltpu.SemaphoreType.DMA((N_LAYERS,)),
            pltpu.SemaphoreType.DMA((N_LAYERS,)),
            pltpu.SemaphoreType.DMA((N_EXCH,)),
            pltpu.SemaphoreType.DMA((N_EXCH,)),
        ],
        compiler_params=pltpu.CompilerParams(
            collective_id=0, vmem_limit_bytes=110 * 1024 * 1024
        ),
    )(x, Win0, Wout0, Win1, Wout1, Win2, Wout2)
